# Initial kernel scaffold; baseline (speedup 1.0000x reference)
import jax
import jax.numpy as jnp
from jax.experimental import pallas as pl


def kernel(inputs, clause_weights):
    raise NotImplementedError("write your pallas kernel here")


# speedup vs baseline: 2.7807x; 2.7807x over previous
import functools

import jax
import jax.numpy as jnp
from jax import lax
from jax.experimental import pallas as pl
from jax.experimental.pallas import tpu as pltpu
from jax.experimental.pallas import tpu_sc as plsc

_N = 100000
_P = 64

_NC = 2
_NS = 16
_NW = _NC * _NS


def _make_sc_kernel(n_rows, rows_chunk, interpret=False):
    nominal = n_rows // _NW
    assert n_rows % 8 == 0 and rows_chunk % 8 == 0
    max_rows_w = max(
        (nominal * (w + 1)) // 8 * 8 - (nominal * w) // 8 * 8 for w in range(_NW)
    )
    n_chunks = -(-max_rows_w // rows_chunk)
    assert rows_chunk <= min(
        (nominal * (w + 1)) // 8 * 8 - (nominal * w) // 8 * 8 for w in range(_NW)
    )

    mesh = plsc.VectorSubcoreMesh(
        core_axis_name="c", subcore_axis_name="s",
        num_cores=_NC, num_subcores=_NS,
    )

    @functools.partial(
        pl.kernel,
        mesh=mesh,
        out_type=jax.ShapeDtypeStruct((n_rows, _P), jnp.float32),
        scratch_types=[
            pltpu.VMEM((rows_chunk, _P), jnp.float32),
            pltpu.VMEM((rows_chunk, _P), jnp.float32),
            pltpu.VMEM((48,), jnp.float32),
            pltpu.VMEM((96,), jnp.float32),
            pltpu.VMEM((_P,), jnp.float32),
        ],
        interpret=interpret,
    )
    def knl(x_hbm, w_hbm, out_hbm, in_v, out_v, wr_v, acc_v, w_v):
        wid = lax.axis_index("c") * _NS + lax.axis_index("s")
        start_w = (nominal * wid) // 8 * 8
        end_w = (nominal * (wid + 1)) // 8 * 8
        pltpu.sync_copy(w_hbm, w_v)
        wpos = [w_v[pl.ds(16 * k, 16)] for k in range(4)]
        wneg = [-w for w in wpos]
        zero16 = jnp.zeros((16,), jnp.float32)

        def row_body(r, _):
            wr_v[pl.ds(0, 16)] = in_v[r, pl.ds(48, 16)]
            wr_v[pl.ds(16, 16)] = in_v[r, pl.ds(0, 16)]
            wr_v[pl.ds(32, 16)] = in_v[r, pl.ds(16, 16)]

            def shread(o):
                if o + 16 <= 64:
                    return in_v[r, pl.ds(o, 16)]
                if o >= 64:
                    return in_v[r, pl.ds(o - 64, 16)]
                return wr_v[pl.ds(o - 48, 16)]

            for k in range(6):
                acc_v[pl.ds(16 * k, 16)] = zero16

            for k in range(4):
                a0 = in_v[r, pl.ds(16 * k, 16)]
                a1 = shread(16 * k + 5)
                a2 = shread(16 * k + 11)
                a3 = shread(16 * k + 17)
                e0 = jnp.exp(-a0)
                e1 = jnp.exp(a1)
                e2 = jnp.exp(-a2)
                e3 = jnp.exp(a3)
                inv = 1.0 / (e0 + e1 + e2 + e3)
                for off, val in (
                    (16 * k, e0 * inv * wneg[k]),
                    (16 * k + 5, e1 * inv * wpos[k]),
                    (16 * k + 11, e2 * inv * wneg[k]),
                    (16 * k + 17, e3 * inv * wpos[k]),
                ):
                    acc_v[pl.ds(off, 16)] = acc_v[pl.ds(off, 16)] + val

            out_v[r, pl.ds(0, 16)] = acc_v[pl.ds(0, 16)] + acc_v[pl.ds(64, 16)]
            out_v[r, pl.ds(16, 16)] = acc_v[pl.ds(16, 16)] + acc_v[pl.ds(80, 16)]
            out_v[r, pl.ds(32, 16)] = acc_v[pl.ds(32, 16)]
            out_v[r, pl.ds(48, 16)] = acc_v[pl.ds(48, 16)]
            return 0

        def chunk_body(i, _):
            r0 = jnp.minimum(start_w + i * rows_chunk, end_w - rows_chunk)
            r0 = pl.multiple_of(r0, 8)
            pltpu.sync_copy(x_hbm.at[pl.ds(r0, rows_chunk)], in_v)
            lax.fori_loop(0, rows_chunk, row_body, 0)
            pltpu.sync_copy(out_v, out_hbm.at[pl.ds(r0, rows_chunk)])
            return 0

        lax.fori_loop(0, n_chunks, chunk_body, 0)

    return knl


_sc_kernel = _make_sc_kernel(_N, 448)


def kernel(inputs, clause_weights):
    return _sc_kernel(inputs, clause_weights)


# speedup vs baseline: 5.0606x; 1.8199x over previous
import functools

import jax
import jax.numpy as jnp
from jax import lax
from jax.experimental import pallas as pl
from jax.experimental.pallas import tpu as pltpu
from jax.experimental.pallas import tpu_sc as plsc

_N = 100000
_P = 64

_NC = 2
_NS = 16
_NW = _NC * _NS
_G = 4


def _make_sc_kernel(n_rows, rows_chunk):
    nominal = n_rows // _NW
    assert n_rows % 8 == 0 and rows_chunk % 8 == 0 and rows_chunk % _G == 0
    sizes = [
        (nominal * (w + 1)) // 8 * 8 - (nominal * w) // 8 * 8 for w in range(_NW)
    ]
    n_chunks = -(-max(sizes) // rows_chunk)
    assert rows_chunk <= min(sizes)

    mesh = plsc.VectorSubcoreMesh(
        core_axis_name="c", subcore_axis_name="s",
        num_cores=_NC, num_subcores=_NS,
    )

    @functools.partial(
        pl.kernel,
        mesh=mesh,
        out_type=jax.ShapeDtypeStruct((n_rows, _P), jnp.float32),
        scratch_types=[
            pltpu.VMEM((rows_chunk, _P), jnp.float32),
            pltpu.VMEM((rows_chunk, _P), jnp.float32),
            pltpu.VMEM((_G, 32), jnp.float32),
            pltpu.VMEM((_G, 96), jnp.float32),
            pltpu.VMEM((_P,), jnp.float32),
        ],
    )
    def knl(x_hbm, w_hbm, out_hbm, in_v, out_v, wr_v, acc_v, w_v):
        wid = lax.axis_index("c") * _NS + lax.axis_index("s")
        start_w = (nominal * wid) // 8 * 8
        end_w = (nominal * (wid + 1)) // 8 * 8
        pltpu.sync_copy(w_hbm, w_v)
        wpos = [w_v[pl.ds(16 * k, 16)] for k in range(4)]
        wneg = [-w for w in wpos]
        zero16 = jnp.zeros((16,), jnp.float32)

        def do_row(r, j):
            a0 = [in_v[r, pl.ds(16 * k, 16)] for k in range(4)]
            wr_v[j, pl.ds(0, 16)] = a0[3]
            wr_v[j, pl.ds(16, 16)] = a0[0]

            def shread(o):
                if o + 16 <= 64:
                    return in_v[r, pl.ds(o, 16)]
                if o >= 64:
                    return in_v[r, pl.ds(o - 64, 16)]
                return wr_v[j, pl.ds(o - 48, 16)]

            d = []
            for k in range(4):
                a1 = shread(16 * k + 5)
                a2 = shread(16 * k + 11)
                a3 = shread(16 * k + 17)
                e0 = jnp.exp(-a0[k])
                e1 = jnp.exp(a1)
                e2 = jnp.exp(-a2)
                e3 = jnp.exp(a3)
                inv = 1.0 / (e0 + e1 + e2 + e3)
                d.append((
                    e0 * inv * wneg[k],
                    e1 * inv * wpos[k],
                    e2 * inv * wneg[k],
                    e3 * inv * wpos[k],
                ))
            for k in range(4):
                acc_v[j, pl.ds(16 * k, 16)] = d[k][0]
            acc_v[j, pl.ds(64, 16)] = zero16
            acc_v[j, pl.ds(80, 16)] = zero16
            for k in range(4):
                plsc.addupdate(acc_v.at[j, pl.ds(16 * k + 5, 16)], d[k][1])
                plsc.addupdate(acc_v.at[j, pl.ds(16 * k + 11, 16)], d[k][2])
                plsc.addupdate(acc_v.at[j, pl.ds(16 * k + 17, 16)], d[k][3])
            out_v[r, pl.ds(0, 16)] = acc_v[j, pl.ds(0, 16)] + acc_v[j, pl.ds(64, 16)]
            out_v[r, pl.ds(16, 16)] = acc_v[j, pl.ds(16, 16)] + acc_v[j, pl.ds(80, 16)]
            out_v[r, pl.ds(32, 16)] = acc_v[j, pl.ds(32, 16)]
            out_v[r, pl.ds(48, 16)] = acc_v[j, pl.ds(48, 16)]

        def group_body(g, _):
            for j in range(_G):
                do_row(g * _G + j, j)
            return 0

        def chunk_body(i, _):
            r0 = jnp.minimum(start_w + i * rows_chunk, end_w - rows_chunk)
            r0 = pl.multiple_of(r0, 8)
            pltpu.sync_copy(x_hbm.at[pl.ds(r0, rows_chunk)], in_v)
            lax.fori_loop(0, rows_chunk // _G, group_body, 0)
            pltpu.sync_copy(out_v, out_hbm.at[pl.ds(r0, rows_chunk)])
            return 0

        lax.fori_loop(0, n_chunks, chunk_body, 0)

    return knl


_sc_kernel = _make_sc_kernel(_N, 448)


def kernel(inputs, clause_weights):
    return _sc_kernel(inputs, clause_weights)
